# baseline (speedup 1.0000x reference)
import functools

import jax
import jax.numpy as jnp
from jax import lax
from jax.experimental import pallas as pl
from jax.experimental.pallas import tpu as pltpu
from jax.experimental.pallas import tpu_sc as plsc

_NC = 2
_NS = 16
_NW = _NC * _NS
_LANES = 16

_HSC = 96
_BR = 8


def _sc_call(HS, W, D):
    RPW = HS // _NW
    CC = 32
    NCH = W // CC
    DH = D // (2 * _LANES)

    mesh = plsc.VectorSubcoreMesh(core_axis_name="c", subcore_axis_name="s")

    @functools.partial(
        pl.kernel,
        out_type=jax.ShapeDtypeStruct((HS * W, D), jnp.float32),
        mesh=mesh,
        scratch_types=[
            pltpu.VMEM((8,), jnp.int32),
            pltpu.VMEM((CC,), jnp.int32),
            pltpu.VMEM((8, D), jnp.float32),
            pltpu.VMEM((CC, D), jnp.float32),
            pltpu.VMEM((CC, D), jnp.float32),
            pltpu.VMEM((CC, D), jnp.float32),
            pltpu.SemaphoreType.DMA,
            pltpu.SemaphoreType.DMA,
            pltpu.SemaphoreType.DMA,
        ],
    )
    def call(ridx_hbm, cidx_hbm, row_hbm, col_hbm, out_hbm,
             ridx_v, cidx_v, row_v, col_v, out_v0, out_v1,
             gsem, sem0, sem1):
        wid = lax.axis_index("c") * _NS + lax.axis_index("s")
        base = wid * RPW
        out_bufs = (out_v0, out_v1)
        sems = (sem0, sem1)
        pltpu.sync_copy(ridx_hbm.at[wid], ridx_v)
        pltpu.async_copy(row_hbm.at[ridx_v], row_v, gsem).wait()
        t = 0
        for c in range(NCH):
            pltpu.sync_copy(cidx_hbm.at[pl.ds(c * CC, CC)], cidx_v)
            pltpu.async_copy(col_hbm.at[cidx_v], col_v, gsem).wait()
            for il in range(RPW):
                b = t % 2
                buf = out_bufs[b]
                sem = sems[b]
                start = (base + il) * W + c * CC
                dst = out_hbm.at[pl.ds(start, CC)]
                if t >= 2:
                    pltpu.make_async_copy(buf, dst, sem).wait()
                for half in range(2):
                    off = half * DH * _LANES
                    rvecs = [row_v[il, pl.ds(off + d * _LANES, _LANES)]
                             for d in range(DH)]

                    def j_body(j, _, buf=buf, off=off, rvecs=rvecs):
                        for d in range(DH):
                            sl = pl.ds(off + d * _LANES, _LANES)
                            buf[j, sl] = col_v[j, sl] + rvecs[d]
                        return ()

                    lax.fori_loop(0, CC, j_body, ())
                pltpu.async_copy(buf, dst, sem)
                t += 1
        for b in range(2):
            pltpu.make_async_copy(
                out_bufs[b], out_hbm.at[pl.ds(base * W, CC)], sems[b]).wait()

    return call


def _tc_body(hw_ref, row_ref, col_ref, out_ref):
    i = pl.program_id(0)
    hm1 = hw_ref[0]
    wm1 = hw_ref[1]
    W = col_ref.shape[0]
    jidx = lax.broadcasted_iota(jnp.int32, (W, 1), 0)
    col_last = col_ref[pl.ds(wm1, 1), :]
    colc = jnp.where(jidx <= wm1, col_ref[...], col_last)
    for r in range(_BR):
        ridx = jnp.minimum(i * _BR + r, hm1)
        rvec = row_ref[pl.ds(ridx, 1), :]
        out_ref[pl.ds(r * W, W), :] = colc + rvec


def _tc_call(HT, W, D, H, hw, row_embed, col_embed):
    return pl.pallas_call(
        _tc_body,
        grid=(HT // _BR,),
        in_specs=[
            pl.BlockSpec(memory_space=pltpu.SMEM),
            pl.BlockSpec((H, D), lambda i: (0, 0)),
            pl.BlockSpec((W, D), lambda i: (0, 0)),
        ],
        out_specs=pl.BlockSpec((_BR * W, D), lambda i: (i, 0)),
        out_shape=jax.ShapeDtypeStruct((HT * W, D), jnp.float32),
    )(hw, row_embed, col_embed)


def kernel(h, w, row_embed, col_embed):
    H, D = row_embed.shape
    W, _ = col_embed.shape
    HT = H - _HSC
    hm1 = jnp.asarray(h, jnp.int32) - 1
    wm1 = jnp.asarray(w, jnp.int32) - 1
    ridx = jnp.minimum(jnp.arange(HT, HT + _HSC, dtype=jnp.int32),
                       hm1).reshape(_NW, -1)
    ridx = jnp.pad(ridx, ((0, 0), (0, 8 - ridx.shape[1])), mode="edge")
    cidx = jnp.minimum(jnp.arange(W, dtype=jnp.int32), wm1)
    sc_out = _sc_call(_HSC, W, D)(ridx, cidx, row_embed, col_embed)
    hw = jnp.stack([hm1, wm1])
    tc_out = _tc_call(HT, W, D, H, hw, row_embed, col_embed)
    return lax.concatenate([tc_out, sc_out], 0)


# speedup vs baseline: 1.9868x; 1.9868x over previous
import functools

import jax
import jax.numpy as jnp
from jax import lax
from jax.experimental import pallas as pl
from jax.experimental.pallas import tpu as pltpu
from jax.experimental.pallas import tpu_sc as plsc

_NC = 2
_NS = 16
_NW = _NC * _NS
_LANES = 16


def _sc_call(H, W, D):
    RPW = H // _NW
    CC = 32
    NCH = W // CC
    DH = D // (2 * _LANES)

    mesh = plsc.VectorSubcoreMesh(core_axis_name="c", subcore_axis_name="s")

    @functools.partial(
        pl.kernel,
        out_type=jax.ShapeDtypeStruct((H * W, D), jnp.float32),
        mesh=mesh,
        scratch_types=[
            pltpu.VMEM((8,), jnp.int32),
            pltpu.VMEM((W,), jnp.int32),
            pltpu.VMEM((8, D), jnp.float32),
            pltpu.VMEM((CC, D), jnp.float32),
            pltpu.VMEM((CC, D), jnp.float32),
            pltpu.VMEM((CC, D), jnp.float32),
            pltpu.VMEM((CC, D), jnp.float32),
            pltpu.SemaphoreType.DMA,
            pltpu.SemaphoreType.DMA,
            pltpu.SemaphoreType.DMA,
            pltpu.SemaphoreType.DMA,
            pltpu.SemaphoreType.DMA,
        ],
    )
    def call(ridx_hbm, cidx_hbm, row_hbm, col_hbm, out_hbm,
             ridx_v, cidx_v, row_v, col_v0, col_v1, out_v0, out_v1,
             rsem, csem0, csem1, sem0, sem1):
        wid = lax.axis_index("c") * _NS + lax.axis_index("s")
        base = wid * RPW
        col_bufs = (col_v0, col_v1)
        csems = (csem0, csem1)
        out_bufs = (out_v0, out_v1)
        sems = (sem0, sem1)
        pltpu.sync_copy(ridx_hbm.at[wid], ridx_v)
        pltpu.sync_copy(cidx_hbm, cidx_v)
        row_cp = pltpu.async_copy(row_hbm.at[ridx_v], row_v, rsem)
        pltpu.async_copy(
            col_hbm.at[cidx_v.at[pl.ds(0, CC)]], col_v0, csem0)
        row_cp.wait()
        for c in range(NCH):
            cb = col_bufs[c % 2]
            if c + 1 < NCH:
                pltpu.async_copy(
                    col_hbm.at[cidx_v.at[pl.ds((c + 1) * CC, CC)]],
                    col_bufs[(c + 1) % 2], csems[(c + 1) % 2])
            pltpu.make_async_copy(col_hbm.at[pl.ds(0, CC)], cb,
                                  csems[c % 2]).wait()

            def il2_body(il2, _, c=c, cb=cb):
                for b in range(2):
                    il = il2 * 2 + b
                    buf = out_bufs[b]
                    sem = sems[b]
                    start = (base + il) * W + c * CC
                    dst = out_hbm.at[pl.ds(start, CC)]
                    if c == 0:
                        @pl.when(il2 > 0)
                        def _():
                            pltpu.make_async_copy(buf, dst, sem).wait()
                    else:
                        pltpu.make_async_copy(buf, dst, sem).wait()
                    for half in range(2):
                        off = half * DH * _LANES
                        rvecs = [row_v[il, pl.ds(off + d * _LANES, _LANES)]
                                 for d in range(DH)]

                        def j_body(j, _, buf=buf, cb=cb, off=off,
                                   rvecs=rvecs):
                            for d in range(DH):
                                sl = pl.ds(off + d * _LANES, _LANES)
                                buf[j, sl] = cb[j, sl] + rvecs[d]
                            return ()

                        lax.fori_loop(0, CC, j_body, ())
                    pltpu.async_copy(buf, dst, sem)
                return ()

            lax.fori_loop(0, RPW // 2, il2_body, ())
        for b in range(2):
            pltpu.make_async_copy(
                out_bufs[b], out_hbm.at[pl.ds(base * W, CC)], sems[b]).wait()

    return call


def kernel(h, w, row_embed, col_embed):
    H, D = row_embed.shape
    W, _ = col_embed.shape
    hm1 = jnp.asarray(h, jnp.int32) - 1
    wm1 = jnp.asarray(w, jnp.int32) - 1
    ridx = jnp.minimum(jnp.arange(H, dtype=jnp.int32), hm1).reshape(_NW, -1)
    ridx = jnp.pad(ridx, ((0, 0), (0, 8 - ridx.shape[1])), mode="edge")
    cidx = jnp.minimum(jnp.arange(W, dtype=jnp.int32), wm1)
    return _sc_call(H, W, D)(ridx, cidx, row_embed, col_embed)
